# Initial kernel scaffold; baseline (speedup 1.0000x reference)
import jax
import jax.numpy as jnp
from jax.experimental import pallas as pl


def kernel(x, embed):
    raise NotImplementedError("write your pallas kernel here")


# speedup vs baseline: 1.9153x; 1.9153x over previous
import functools

import jax
import jax.numpy as jnp
from jax import lax
from jax.experimental import pallas as pl
from jax.experimental.pallas import tpu as pltpu

CODEBOOK = 1024
DIM = 256
N_TOKENS = 16 * 576
BLK = 768


def _vq_kernel(x_ref, embed_ref, q_ref, idx_ref, counts_ref, perp_ref):
    i = pl.program_id(0)
    nsteps = pl.num_programs(0)

    x = x_ref[...]
    emb = embed_ref[...]

    dot = lax.dot_general(x, emb, (((1,), (1,)), ((), ())),
                          preferred_element_type=jnp.float32)
    emb_sq = jnp.sum(emb * emb, axis=1)
    dist = 2.0 * dot - emb_sq[None, :]

    idx = jnp.argmax(dist, axis=1).astype(jnp.int32)
    idx_ref[...] = idx.reshape(1, 1, BLK)

    iota_k = lax.broadcasted_iota(jnp.int32, (BLK, CODEBOOK), 1)
    onehot = (iota_k == idx[:, None]).astype(jnp.float32)

    q_ref[...] = lax.dot_general(onehot, emb, (((1,), (0,)), ((), ())),
                                 preferred_element_type=jnp.float32)

    @pl.when(i == 0)
    def _init():
        counts_ref[...] = jnp.zeros_like(counts_ref)

    counts_ref[...] += jnp.sum(onehot, axis=0, keepdims=True)

    @pl.when(i == nsteps - 1)
    def _fin():
        probs = counts_ref[...] / float(N_TOKENS)
        ent = jnp.sum(probs * jnp.log(probs + 1e-10), keepdims=True)
        perp_ref[...] = jnp.exp(-ent).reshape(1, 1)


@jax.jit
def kernel(x, embed):
    shape = x.shape
    flat = x.reshape(-1, DIM)
    grid = N_TOKENS // BLK

    q, idx3, counts, perp = pl.pallas_call(
        _vq_kernel,
        grid=(grid,),
        in_specs=[
            pl.BlockSpec((BLK, DIM), lambda i: (i, 0)),
            pl.BlockSpec((CODEBOOK, DIM), lambda i: (0, 0)),
        ],
        out_specs=[
            pl.BlockSpec((BLK, DIM), lambda i: (i, 0)),
            pl.BlockSpec((1, 1, BLK), lambda i: (i, 0, 0)),
            pl.BlockSpec((1, CODEBOOK), lambda i: (0, 0)),
            pl.BlockSpec((1, 1), lambda i: (0, 0)),
        ],
        out_shape=[
            jax.ShapeDtypeStruct((N_TOKENS, DIM), jnp.float32),
            jax.ShapeDtypeStruct((grid, 1, BLK), jnp.int32),
            jax.ShapeDtypeStruct((1, CODEBOOK), jnp.float32),
            jax.ShapeDtypeStruct((1, 1), jnp.float32),
        ],
    )(flat, embed)

    quantize = q.reshape(shape)
    embed_ind = idx3.reshape(shape[:-1])
    perplexity = perp.reshape(())
    return quantize, embed_ind, perplexity
